# baseline (speedup 1.0000x reference)
import jax
import jax.numpy as jnp
from jax import lax
from jax.experimental import pallas as pl
from jax.experimental.pallas import tpu as pltpu
from jax.experimental.pallas import tpu_sc as plsc

QF = 1024
L = 16
NC, NS = 2, 16
NW = NC * NS
WIN = 1024
STAGE = 8
D = 64


def _body(n_idx, n_bags, idx_hbm, off_hbm, wq_hbm, wr_hbm, out_hbm,
          wq_v, wr_v, off_v, off2_v, win_v, stage_v):
    bags_per_w = n_bags // NW
    c = lax.axis_index("c")
    s = lax.axis_index("s")
    w = s * NC + c
    b0 = pl.multiple_of(w * bags_per_w, bags_per_w)

    pltpu.sync_copy(wq_hbm, wq_v)
    pltpu.sync_copy(wr_hbm, wr_v)
    pltpu.sync_copy(off_hbm.at[pl.ds(b0, bags_per_w)],
                    off_v.at[pl.ds(0, bags_per_w)])

    @pl.when(w < NW - 1)
    def _():
        pltpu.sync_copy(off_hbm.at[pl.ds(b0 + bags_per_w, 8)],
                        off2_v.at[pl.ds(0, 8)])

    i_end_w = jnp.where(w < NW - 1, off2_v[pl.ds(0, L)][0], n_idx)
    zero = jnp.zeros((L,), jnp.float32)

    CH = WIN // 2

    def bag_body(b, win_base):
        s_i = off_v[pl.ds(b, L)][0]
        nxt = off_v[pl.ds(b + 1, L)][0]
        e_i = jnp.where(b < bags_per_w - 1, nxt, i_end_w)

        def chunk_body(c, st):
            wb, a0, a1, a2, a3 = st
            cs = s_i + lax.shift_left(c, 9)
            ce = jnp.minimum(cs + CH, e_i)
            need = ce > wb + WIN
            nwb = pl.multiple_of(
                jnp.where(need,
                          jnp.minimum(lax.bitwise_and(cs, -8), n_idx - WIN),
                          wb), 8)

            @pl.when(need)
            def _():
                pltpu.sync_copy(idx_hbm.at[pl.ds(nwb, WIN)],
                                win_v.at[pl.ds(0, WIN)])

            base = cs - nwb
            ngrp = lax.shift_right_logical(ce - cs, 4)

            def gbody(g, accs):
                v = win_v[pl.ds(base + lax.shift_left(g, 4), L)]
                qbs, rbs = [], []
                for j in range(16):
                    ix = v[j]
                    qbs.append(lax.shift_left(
                        lax.shift_right_logical(ix, 10), 6))
                    rbs.append(lax.shift_left(
                        lax.bitwise_and(ix, QF - 1), 6))
                out = list(accs)
                for k in range(4):
                    rows = [wq_v[pl.ds(qbs[j] + k * L, L)]
                            + wr_v[pl.ds(rbs[j] + k * L, L)]
                            for j in range(16)]
                    t = [rows[2 * j] + rows[2 * j + 1] for j in range(8)]
                    u = [t[2 * j] + t[2 * j + 1] for j in range(4)]
                    s01 = u[0] + u[1]
                    s23 = u[2] + u[3]
                    out[k] = out[k] + (s01 + s23)
                return tuple(out)

            def ibody(i, accs):
                a0, a1, a2, a3 = accs
                ix = win_v[pl.ds(i - nwb, L)][0]
                qb = lax.shift_left(lax.shift_right_logical(ix, 10), 6)
                rb = lax.shift_left(lax.bitwise_and(ix, QF - 1), 6)
                a0 = a0 + (wq_v[pl.ds(qb + 0 * L, L)]
                           + wr_v[pl.ds(rb + 0 * L, L)])
                a1 = a1 + (wq_v[pl.ds(qb + 1 * L, L)]
                           + wr_v[pl.ds(rb + 1 * L, L)])
                a2 = a2 + (wq_v[pl.ds(qb + 2 * L, L)]
                           + wr_v[pl.ds(rb + 2 * L, L)])
                a3 = a3 + (wq_v[pl.ds(qb + 3 * L, L)]
                           + wr_v[pl.ds(rb + 3 * L, L)])
                return (a0, a1, a2, a3)

            accs = lax.fori_loop(0, ngrp, gbody, (a0, a1, a2, a3))
            accs = lax.fori_loop(cs + lax.shift_left(ngrp, 4), ce,
                                 ibody, accs)
            a0, a1, a2, a3 = accs
            return (nwb, a0, a1, a2, a3)

        nch = lax.shift_right_logical(e_i - s_i + CH - 1, 9)
        st = lax.fori_loop(0, nch, chunk_body,
                           (win_base, zero, zero, zero, zero))
        win_base, a0, a1, a2, a3 = st

        sd = lax.bitwise_and(b, STAGE - 1) * D
        stage_v[pl.ds(sd + 0 * L, L)] = a0
        stage_v[pl.ds(sd + 1 * L, L)] = a1
        stage_v[pl.ds(sd + 2 * L, L)] = a2
        stage_v[pl.ds(sd + 3 * L, L)] = a3

        @pl.when(sd == (STAGE - 1) * D)
        def _():
            dst = pl.multiple_of((b0 + b - (STAGE - 1)) * D, STAGE * D)
            pltpu.sync_copy(stage_v, out_hbm.at[pl.ds(dst, STAGE * D)])

        return win_base

    lax.fori_loop(0, bags_per_w, bag_body, jnp.int32(-(1 << 30)))


def kernel(indices, offsets, Wq, Wr):
    n_idx = indices.shape[0]
    n_bags = offsets.shape[0]
    qn, d = Wq.shape
    qf = Wr.shape[0]
    indices = indices.astype(jnp.int32)
    offsets = offsets.astype(jnp.int32)
    wq_flat = Wq.reshape(-1)
    wr_flat = Wr.reshape(-1)

    mesh = plsc.VectorSubcoreMesh(core_axis_name="c", subcore_axis_name="s")
    body = lambda *refs: _body(n_idx, n_bags, *refs)
    fn = pl.kernel(
        body,
        out_type=jax.ShapeDtypeStruct((n_bags * d,), jnp.float32),
        mesh=mesh,
        scratch_types=[
            pltpu.VMEM((qn * d,), jnp.float32),
            pltpu.VMEM((qf * d,), jnp.float32),
            pltpu.VMEM((n_bags // NW + 24, ), jnp.int32),
            pltpu.VMEM((16,), jnp.int32),
            pltpu.VMEM((WIN + 16,), jnp.int32),
            pltpu.VMEM((STAGE * d,), jnp.float32),
        ],
    )
    return fn(indices, offsets, wq_flat, wr_flat).reshape(n_bags, d)


# speedup vs baseline: 1.0413x; 1.0413x over previous
import jax
import jax.numpy as jnp
from jax import lax
from jax.experimental import pallas as pl
from jax.experimental.pallas import tpu as pltpu
from jax.experimental.pallas import tpu_sc as plsc

QF = 1024
L = 16
NC, NS = 2, 16
NW = NC * NS
WIN = 1024
STAGE = 8
D = 64


def _body(n_idx, n_bags, idx_hbm, off_hbm, wq_hbm, wr_hbm, out_hbm,
          wq_v, wr_v, off_v, off2_v, win_v, stage_v):
    bags_per_w = n_bags // NW
    c = lax.axis_index("c")
    s = lax.axis_index("s")
    w = s * NC + c
    b0 = pl.multiple_of(w * bags_per_w, bags_per_w)

    pltpu.sync_copy(wq_hbm, wq_v)
    pltpu.sync_copy(wr_hbm, wr_v)
    pltpu.sync_copy(off_hbm.at[pl.ds(b0, bags_per_w)],
                    off_v.at[pl.ds(0, bags_per_w)])

    @pl.when(w < NW - 1)
    def _():
        pltpu.sync_copy(off_hbm.at[pl.ds(b0 + bags_per_w, 8)],
                        off2_v.at[pl.ds(0, 8)])

    i_end_w = jnp.where(w < NW - 1, off2_v[pl.ds(0, L)][0], n_idx)
    zero = jnp.zeros((L,), jnp.float32)

    CH = WIN // 2

    def bag_body(b, win_base):
        s_i = off_v[pl.ds(b, L)][0]
        nxt = off_v[pl.ds(b + 1, L)][0]
        e_i = jnp.where(b < bags_per_w - 1, nxt, i_end_w)

        def chunk_body(c, st):
            wb, a0, a1, a2, a3 = st
            cs = s_i + lax.shift_left(c, 9)
            ce = jnp.minimum(cs + CH, e_i)
            need = ce > wb + WIN
            nwb = pl.multiple_of(
                jnp.where(need,
                          jnp.minimum(lax.bitwise_and(cs, -8), n_idx - WIN),
                          wb), 8)

            @pl.when(need)
            def _():
                pltpu.sync_copy(idx_hbm.at[pl.ds(nwb, WIN)],
                                win_v.at[pl.ds(0, WIN)])

            base = cs - nwb
            ngrp = lax.shift_right_logical(ce - cs, 3)

            def gbody(g, accs):
                v = win_v[pl.ds(base + lax.shift_left(g, 3), L)]
                qbs, rbs = [], []
                for j in range(8):
                    ix = v[j]
                    qbs.append(lax.shift_left(
                        lax.shift_right_logical(ix, 10), 6))
                    rbs.append(lax.shift_left(
                        lax.bitwise_and(ix, QF - 1), 6))
                out = list(accs)
                for k in range(4):
                    rows = [wq_v[pl.ds(qbs[j] + k * L, L)]
                            + wr_v[pl.ds(rbs[j] + k * L, L)]
                            for j in range(8)]
                    t01 = rows[0] + rows[1]
                    t23 = rows[2] + rows[3]
                    t45 = rows[4] + rows[5]
                    t67 = rows[6] + rows[7]
                    out[k] = out[k] + ((t01 + t23) + (t45 + t67))
                return tuple(out)

            def ibody(i, accs):
                a0, a1, a2, a3 = accs
                ix = win_v[pl.ds(i - nwb, L)][0]
                qb = lax.shift_left(lax.shift_right_logical(ix, 10), 6)
                rb = lax.shift_left(lax.bitwise_and(ix, QF - 1), 6)
                a0 = a0 + (wq_v[pl.ds(qb + 0 * L, L)]
                           + wr_v[pl.ds(rb + 0 * L, L)])
                a1 = a1 + (wq_v[pl.ds(qb + 1 * L, L)]
                           + wr_v[pl.ds(rb + 1 * L, L)])
                a2 = a2 + (wq_v[pl.ds(qb + 2 * L, L)]
                           + wr_v[pl.ds(rb + 2 * L, L)])
                a3 = a3 + (wq_v[pl.ds(qb + 3 * L, L)]
                           + wr_v[pl.ds(rb + 3 * L, L)])
                return (a0, a1, a2, a3)

            accs = lax.fori_loop(0, ngrp, gbody, (a0, a1, a2, a3))
            accs = lax.fori_loop(cs + lax.shift_left(ngrp, 3), ce,
                                 ibody, accs)
            a0, a1, a2, a3 = accs
            return (nwb, a0, a1, a2, a3)

        nch = lax.shift_right_logical(e_i - s_i + CH - 1, 9)
        st = lax.fori_loop(0, nch, chunk_body,
                           (win_base, zero, zero, zero, zero))
        win_base, a0, a1, a2, a3 = st

        sd = lax.bitwise_and(b, STAGE - 1) * D
        stage_v[pl.ds(sd + 0 * L, L)] = a0
        stage_v[pl.ds(sd + 1 * L, L)] = a1
        stage_v[pl.ds(sd + 2 * L, L)] = a2
        stage_v[pl.ds(sd + 3 * L, L)] = a3

        @pl.when(sd == (STAGE - 1) * D)
        def _():
            dst = pl.multiple_of((b0 + b - (STAGE - 1)) * D, STAGE * D)
            pltpu.sync_copy(stage_v, out_hbm.at[pl.ds(dst, STAGE * D)])

        return win_base

    lax.fori_loop(0, bags_per_w, bag_body, jnp.int32(-(1 << 30)))


def kernel(indices, offsets, Wq, Wr):
    n_idx = indices.shape[0]
    n_bags = offsets.shape[0]
    qn, d = Wq.shape
    qf = Wr.shape[0]
    indices = indices.astype(jnp.int32)
    offsets = offsets.astype(jnp.int32)
    wq_flat = Wq.reshape(-1)
    wr_flat = Wr.reshape(-1)

    mesh = plsc.VectorSubcoreMesh(core_axis_name="c", subcore_axis_name="s")
    body = lambda *refs: _body(n_idx, n_bags, *refs)
    fn = pl.kernel(
        body,
        out_type=jax.ShapeDtypeStruct((n_bags * d,), jnp.float32),
        mesh=mesh,
        scratch_types=[
            pltpu.VMEM((qn * d,), jnp.float32),
            pltpu.VMEM((qf * d,), jnp.float32),
            pltpu.VMEM((n_bags // NW + 24, ), jnp.int32),
            pltpu.VMEM((16,), jnp.int32),
            pltpu.VMEM((WIN + 16,), jnp.int32),
            pltpu.VMEM((STAGE * d,), jnp.float32),
        ],
    )
    return fn(indices, offsets, wq_flat, wr_flat).reshape(n_bags, d)
